# baseline (speedup 1.0000x reference)
import functools

import jax
import jax.numpy as jnp
from jax import lax
from jax.experimental import pallas as pl
from jax.experimental.pallas import tpu as pltpu
from jax.experimental.pallas import tpu_sc as plsc

Q = 32
D = 64
V = 1_000_000
B = 1024
L = 128
N_TOK = B * L

_NC, _NS = 2, 16
_TPT = N_TOK // _NS
_FPC = D // _NC
_FGRP = 8
_IDX_MINOR = 128
_NROW = _TPT // _IDX_MINOR
_W = 499712
_WS = _W // _NS
_TAIL = V - 2 * _W
_TAIL_PAD = 640
_SENT = 2**31 - 1


def _sc_gather_body(wt_hbm, tail_hbm, idx_hbm, out_hbm,
                    row_sp, idx_a, idx_b, grp_v, sem, sem2):
    cid = lax.axis_index("c")
    sid = lax.axis_index("s")
    pltpu.sync_copy(idx_hbm.at[sid], idx_a)

    def prep(t, _):
        r = t // (_IDX_MINOR // 16)
        c = t % (_IDX_MINOR // 16)
        v = idx_a[r, pl.ds(c * 16, 16)]
        sent = jnp.full((16,), _SENT, jnp.int32)
        idx_b[r, pl.ds(c * 16, 16)] = jnp.where(v >= _W, v - _W, sent)
        idx_a[r, pl.ds(c * 16, 16)] = jnp.where(v < _W, v, sent)
        return ()

    lax.fori_loop(0, _NROW * (_IDX_MINOR // 16), prep, (), unroll=False)

    def gather_window(idx_ref, j8):
        def mk(i):
            return (
                row_sp.at[plsc.Indices(idx_ref.at[i], ignored_value=_SENT)],
                grp_v.at[j8, pl.ds(i * _IDX_MINOR, _IDX_MINOR)],
                sem,
            )

        def fire(k, _):
            for i in range(8):
                pltpu.async_copy(*mk(k * 8 + i))
            return ()

        lax.fori_loop(0, _NROW // 8, fire, (), unroll=False)
        pltpu.make_async_copy(
            wt_hbm.at[0].at[pl.ds(0, _TPT)], grp_v.at[j8], sem
        ).wait()

    def stage(row_hbm, src_base):
        sub = _WS // 4
        copies = [
            pltpu.async_copy(
                row_hbm.at[pl.ds(pl.multiple_of(src_base + sid * _WS + q * sub,
                                                128), sub)],
                row_sp.at[pl.ds(pl.multiple_of(sid * _WS + q * sub, 128), sub)],
                sem2,
            )
            for q in range(4)
        ]
        for c in copies:
            c.wait()

    def feat_group(g, _):
        def feat(j8, _):
            jj = cid * _FPC + g * _FGRP + j8
            row_hbm = wt_hbm.at[jj]
            stage(row_hbm, 0)
            plsc.subcore_barrier()
            gather_window(idx_a, j8)
            plsc.subcore_barrier()
            stage(row_hbm, _W)

            @pl.when(sid == 0)
            def _():
                pltpu.sync_copy(tail_hbm.at[jj],
                                row_sp.at[pl.ds(_W, _TAIL_PAD)])

            plsc.subcore_barrier()
            gather_window(idx_b, j8)
            plsc.subcore_barrier()
            return ()

        lax.fori_loop(0, _FGRP, feat, (), unroll=False)
        row_start = pl.multiple_of(cid * _FPC + g * _FGRP, _FGRP)
        pltpu.sync_copy(
            grp_v,
            out_hbm.at[pl.ds(row_start, _FGRP), pl.ds(sid * _TPT, _TPT)],
        )
        return ()

    lax.fori_loop(0, _FPC // _FGRP, feat_group, (), unroll=False)


@functools.cache
def _sc_gather():
    return functools.partial(
        pl.kernel,
        out_type=jax.ShapeDtypeStruct((D, N_TOK), jnp.float32),
        mesh=plsc.VectorSubcoreMesh(core_axis_name="c", subcore_axis_name="s"),
        scratch_types=[
            pltpu.VMEM_SHARED((_W + _TAIL_PAD,), jnp.float32),
            pltpu.VMEM((_NROW, _IDX_MINOR), jnp.int32),
            pltpu.VMEM((_NROW, _IDX_MINOR), jnp.int32),
            pltpu.VMEM((_FGRP, _TPT), jnp.float32),
            pltpu.SemaphoreType.DMA,
            pltpu.SemaphoreType.DMA,
        ],
    )(_sc_gather_body)


_BLK = 8192


def _tc_dist_body(x_ref, t_ref, o_ref):
    x = x_ref[...]
    t = t_ref[...]
    a2 = jnp.sum(x * x, axis=1)
    ones = jnp.ones((1, D), jnp.float32)
    b2 = lax.dot_general(ones, t * t, (((1,), (0,)), ((), ())),
                         preferred_element_type=jnp.float32)
    ab = lax.dot_general(x, t, (((1,), (0,)), ((), ())),
                         preferred_element_type=jnp.float32)
    d2 = b2 + a2[:, None] - 2.0 * ab
    s = 0.0
    for i in range(_BLK // L):
        m = jnp.min(d2[:, i * L:(i + 1) * L], axis=1)
        s = s + jnp.sum(jnp.sqrt(jnp.maximum(m, 0.0)))

    @pl.when(pl.program_id(0) == 0)
    def _():
        o_ref[0, 0] = 0.0

    o_ref[0, 0] += s


def _tc_dist(x, targets_t):
    return pl.pallas_call(
        _tc_dist_body,
        grid=(N_TOK // _BLK,),
        in_specs=[
            pl.BlockSpec((Q, D), lambda i: (0, 0)),
            pl.BlockSpec((D, _BLK), lambda i: (0, i)),
        ],
        out_specs=pl.BlockSpec(memory_space=pltpu.SMEM),
        out_shape=jax.ShapeDtypeStruct((1, 1), jnp.float32),
    )(x, targets_t)


def kernel(image_features, input_ids, embedding_weight):
    wt = embedding_weight.T
    tail = jnp.pad(wt[:, 2 * _W:], ((0, 0), (0, _TAIL_PAD - _TAIL)))
    idx = input_ids.reshape(_NS, _TPT // _IDX_MINOR, _IDX_MINOR)
    gathered_t = _sc_gather()(wt, tail, idx)
    s = _tc_dist(image_features, gathered_t)
    return s[0, 0] / jnp.float32(B * Q)


# speedup vs baseline: 1.0253x; 1.0253x over previous
import functools

import jax
import jax.numpy as jnp
from jax import lax
from jax.experimental import pallas as pl
from jax.experimental.pallas import tpu as pltpu
from jax.experimental.pallas import tpu_sc as plsc

Q = 32
D = 64
V = 1_000_000
B = 1024
L = 128
N_TOK = B * L

_NC, _NS = 2, 16
_TPT = N_TOK // _NS
_FPC = D // _NC
_FGRP = 8
_IDX_MINOR = 128
_NROW = _TPT // _IDX_MINOR
_W = 499712
_WS = _W // _NS
_TAIL = V - 2 * _W
_TAIL_PAD = 640
_SENT = 2**31 - 1


def _sc_gather_body(wt_hbm, tail_hbm, idx_hbm, out_hbm,
                    row_sp, idx_a, idx_b, grp_v, sem):
    cid = lax.axis_index("c")
    sid = lax.axis_index("s")
    pltpu.sync_copy(idx_hbm.at[sid], idx_a)

    def prep(t, _):
        r = t // (_IDX_MINOR // 16)
        c = t % (_IDX_MINOR // 16)
        v = idx_a[r, pl.ds(c * 16, 16)]
        sent = jnp.full((16,), _SENT, jnp.int32)
        idx_b[r, pl.ds(c * 16, 16)] = jnp.where(v >= _W, v - _W, sent)
        idx_a[r, pl.ds(c * 16, 16)] = jnp.where(v < _W, v, sent)
        return ()

    lax.fori_loop(0, _NROW * (_IDX_MINOR // 16), prep, (), unroll=False)

    def gather_window(idx_ref, j8):
        def mk(i):
            return (
                row_sp.at[plsc.Indices(idx_ref.at[i], ignored_value=_SENT)],
                grp_v.at[j8, pl.ds(i * _IDX_MINOR, _IDX_MINOR)],
                sem,
            )

        def fire(k, _):
            for i in range(8):
                pltpu.async_copy(*mk(k * 8 + i))
            return ()

        lax.fori_loop(0, _NROW // 8, fire, (), unroll=False)
        pltpu.make_async_copy(
            wt_hbm.at[0].at[pl.ds(0, _TPT)], grp_v.at[j8], sem
        ).wait()

    def stage(row_hbm, src_base):
        off = pl.multiple_of(sid * _WS, 128)
        src = pl.multiple_of(src_base + sid * _WS, 128)
        pltpu.sync_copy(row_hbm.at[pl.ds(src, _WS)],
                        row_sp.at[pl.ds(off, _WS)])

    def feat_group(g, _):
        def feat(j8, _):
            jj = cid * _FPC + g * _FGRP + j8
            row_hbm = wt_hbm.at[jj]
            stage(row_hbm, 0)
            plsc.subcore_barrier()
            gather_window(idx_a, j8)
            plsc.subcore_barrier()
            stage(row_hbm, _W)

            @pl.when(sid == 0)
            def _():
                pltpu.sync_copy(tail_hbm.at[jj],
                                row_sp.at[pl.ds(_W, _TAIL_PAD)])

            plsc.subcore_barrier()
            gather_window(idx_b, j8)
            plsc.subcore_barrier()
            return ()

        lax.fori_loop(0, _FGRP, feat, (), unroll=False)
        row_start = pl.multiple_of(cid * _FPC + g * _FGRP, _FGRP)
        pltpu.sync_copy(
            grp_v,
            out_hbm.at[pl.ds(row_start, _FGRP), pl.ds(sid * _TPT, _TPT)],
        )
        return ()

    lax.fori_loop(0, _FPC // _FGRP, feat_group, (), unroll=False)


@functools.cache
def _sc_gather():
    return functools.partial(
        pl.kernel,
        out_type=jax.ShapeDtypeStruct((D, N_TOK), jnp.float32),
        mesh=plsc.VectorSubcoreMesh(core_axis_name="c", subcore_axis_name="s"),
        scratch_types=[
            pltpu.VMEM_SHARED((_W + _TAIL_PAD,), jnp.float32),
            pltpu.VMEM((_NROW, _IDX_MINOR), jnp.int32),
            pltpu.VMEM((_NROW, _IDX_MINOR), jnp.int32),
            pltpu.VMEM((_FGRP, _TPT), jnp.float32),
            pltpu.SemaphoreType.DMA,
        ],
    )(_sc_gather_body)


_BLK = 8192


def _tc_dist_body(x_ref, t_ref, o_ref):
    x = x_ref[...]
    t = t_ref[...]
    a2 = jnp.sum(x * x, axis=1)
    ones = jnp.ones((1, D), jnp.float32)
    b2 = lax.dot_general(ones, t * t, (((1,), (0,)), ((), ())),
                         preferred_element_type=jnp.float32)
    ab = lax.dot_general(x, t, (((1,), (0,)), ((), ())),
                         preferred_element_type=jnp.float32)
    d2 = b2 + a2[:, None] - 2.0 * ab
    s = 0.0
    for i in range(_BLK // L):
        m = jnp.min(d2[:, i * L:(i + 1) * L], axis=1)
        s = s + jnp.sum(jnp.sqrt(jnp.maximum(m, 0.0)))

    @pl.when(pl.program_id(0) == 0)
    def _():
        o_ref[0, 0] = 0.0

    o_ref[0, 0] += s


def _tc_dist(x, targets_t):
    return pl.pallas_call(
        _tc_dist_body,
        grid=(N_TOK // _BLK,),
        in_specs=[
            pl.BlockSpec((Q, D), lambda i: (0, 0)),
            pl.BlockSpec((D, _BLK), lambda i: (0, i)),
        ],
        out_specs=pl.BlockSpec(memory_space=pltpu.SMEM),
        out_shape=jax.ShapeDtypeStruct((1, 1), jnp.float32),
    )(x, targets_t)


def kernel(image_features, input_ids, embedding_weight):
    wt = embedding_weight.T
    tail = jnp.pad(wt[:, 2 * _W:], ((0, 0), (0, _TAIL_PAD - _TAIL)))
    idx = input_ids.reshape(_NS, _TPT // _IDX_MINOR, _IDX_MINOR)
    gathered_t = _sc_gather()(wt, tail, idx)
    s = _tc_dist(image_features, gathered_t)
    return s[0, 0] / jnp.float32(B * Q)
